# Initial kernel scaffold; baseline (speedup 1.0000x reference)
import jax
import jax.numpy as jnp
from jax.experimental import pallas as pl


def kernel(x, edge_index, W, b):
    raise NotImplementedError("write your pallas kernel here")


# speedup vs baseline: 8.2429x; 8.2429x over previous
import functools

import jax
import jax.numpy as jnp
from jax import lax
from jax.experimental import pallas as pl
from jax.experimental.pallas import tpu as pltpu
from jax.experimental.pallas import tpu_sc as plsc

N = 10000
D = 128
E = 320000
R_PAD = 10240
D_PAD = 144
NW = 32
CHUNK = 128
CPT = 80
E_PAD = NW * CPT * CHUNK
ROWS_PER_TILE = R_PAD // 16
KB = ROWS_PER_TILE // CHUNK


def _sc_aggregate(x_aug, src3, dst3, zrows):
    mesh = plsc.VectorSubcoreMesh(core_axis_name="c", subcore_axis_name="s")

    @functools.partial(
        pl.kernel,
        mesh=mesh,
        compiler_params=pltpu.CompilerParams(use_tc_tiling_on_sc=False),
        out_type=jax.ShapeDtypeStruct((2 * R_PAD, D_PAD), jnp.float32),
        scratch_types=[
            pltpu.VMEM((CPT, CHUNK), jnp.int32),
            pltpu.VMEM((CPT, CHUNK), jnp.int32),
            pltpu.VMEM((CHUNK, D_PAD), jnp.float32),
            pltpu.VMEM_SHARED((R_PAD, D_PAD), jnp.float32),
            pltpu.SemaphoreType.DMA,
        ],
    )
    def body(x_hbm, src_hbm, dst_hbm, z_hbm, out_hbm,
             src_v, dst_v, rows_v, acc_sh, sem):
        cid = lax.axis_index("c")
        sid = lax.axis_index("s")
        w = cid * 16 + sid
        pltpu.sync_copy(src_hbm.at[w], src_v)
        pltpu.sync_copy(dst_hbm.at[w], dst_v)
        pltpu.sync_copy(z_hbm, rows_v)
        for k in range(KB):
            pltpu.sync_copy(
                rows_v, acc_sh.at[pl.ds(sid * ROWS_PER_TILE + k * CHUNK, CHUNK)])
        plsc.subcore_barrier()

        def step(j, carry):
            pltpu.async_copy(x_hbm.at[src_v.at[j]], rows_v, sem).wait()
            pltpu.sync_copy(rows_v, acc_sh.at[dst_v.at[j]], add=True)
            return carry

        lax.fori_loop(0, CPT, step, 0)
        plsc.subcore_barrier()
        for k in range(KB):
            r = sid * ROWS_PER_TILE + k * CHUNK
            pltpu.sync_copy(acc_sh.at[pl.ds(r, CHUNK)],
                            out_hbm.at[pl.ds(cid * R_PAD + r, CHUNK)])

    return body(x_aug, src3, dst3, zrows)


def _tc_finish(parts, W, b):
    w_pad = jnp.concatenate([W, jnp.zeros((D_PAD - D, D), jnp.float32)], axis=0)
    w_deg = jnp.zeros((D_PAD, D), jnp.float32).at[D].set(1.0)
    b2 = b.reshape(1, D)

    def body(p0_ref, p1_ref, w_ref, wd_ref, b_ref, o_ref):
        s = p0_ref[...] + p1_ref[...]
        num = jnp.dot(s, w_ref[...], preferred_element_type=jnp.float32)
        den = jnp.dot(s, wd_ref[...], preferred_element_type=jnp.float32)
        o_ref[...] = num / jnp.maximum(den, 1.0) + b_ref[...]

    blk = 1024
    grid = R_PAD // blk
    out = pl.pallas_call(
        body,
        grid=(grid,),
        in_specs=[
            pl.BlockSpec((blk, D_PAD), lambda i: (i, 0)),
            pl.BlockSpec((blk, D_PAD), lambda i, g=grid: (i + g, 0)),
            pl.BlockSpec((D_PAD, D), lambda i: (0, 0)),
            pl.BlockSpec((D_PAD, D), lambda i: (0, 0)),
            pl.BlockSpec((1, D), lambda i: (0, 0)),
        ],
        out_specs=pl.BlockSpec((blk, D), lambda i: (i, 0)),
        out_shape=jax.ShapeDtypeStruct((R_PAD, D), jnp.float32),
    )(parts, parts, w_pad, w_deg, b2)
    return out[:N]


def kernel(x, edge_index, W, b):
    ones = jnp.ones((N, 1), jnp.float32)
    zc = jnp.zeros((N, D_PAD - D - 1), jnp.float32)
    x_aug = jnp.concatenate([x, ones, zc], axis=1)
    x_aug = jnp.concatenate(
        [x_aug, jnp.zeros((R_PAD - N, D_PAD), jnp.float32)], axis=0)
    npad = E_PAD - E
    pad_idx = N + (jnp.arange(npad, dtype=jnp.int32) % (R_PAD - N))
    src = jnp.concatenate([edge_index[0], pad_idx])
    dst = jnp.concatenate([edge_index[1], pad_idx])
    src3 = src.reshape(NW, CPT, CHUNK)
    dst3 = dst.reshape(NW, CPT, CHUNK)
    zrows = jnp.zeros((CHUNK, D_PAD), jnp.float32)
    parts = _sc_aggregate(x_aug, src3, dst3, zrows)
    return _tc_finish(parts, W, b)
